# baseline (device time: 54109 ns/iter reference)
import jax
import jax.numpy as jnp
from jax import lax
from jax.experimental import pallas as pl
from jax.experimental.pallas import tpu as pltpu

N_DEV = 4
B = 512
D = 256
HS = 512
NL = 3
HB = B // 2


def kernel(x, Win0, Wout0, Win1, Wout1, Win2, Wout2):
    def body(x_ref, win0_ref, wout0_ref, win1_ref, wout1_ref, win2_ref,
             wout2_ref, out_ref,
             packed_own, recv_left, recv_right, recv_diag,
             s_own_send, s_own_recv, s_rel_send, s_rel_recv,
             s_ag_own_send, s_ag_own_recv, s_ag_rel_send, s_ag_rel_recv):
        me = lax.axis_index("i")
        right = lax.rem(me + 1, N_DEV)
        left = lax.rem(me + 3, N_DEV)

        barrier = pltpu.get_barrier_semaphore()
        for nbr in (left, right):
            pl.semaphore_signal(barrier, inc=1, device_id=(nbr,),
                                device_id_type=pl.DeviceIdType.MESH)
        pl.semaphore_wait(barrier, 2)

        win_refs = (win0_ref, win1_ref, win2_ref)
        wout_refs = (wout0_ref, wout1_ref, wout2_ref)

        d_toL, d_toR = [], []
        for l in range(NL):
            packed_own[l, 0:D, :] = win_refs[l][...].astype(jnp.bfloat16)
            packed_own[l, D:D + HS // 2, 0:D] = (
                wout_refs[l][0:HS // 2, :].astype(jnp.bfloat16))
            packed_own[l, D:D + HS // 2, D:2 * D] = (
                wout_refs[l][HS // 2:HS, :].astype(jnp.bfloat16))
            dL = pltpu.make_async_remote_copy(
                src_ref=packed_own.at[l], dst_ref=recv_right.at[l],
                send_sem=s_own_send.at[l, 0], recv_sem=s_own_recv.at[l, 1],
                device_id=(left,), device_id_type=pl.DeviceIdType.MESH)
            dR = pltpu.make_async_remote_copy(
                src_ref=packed_own.at[l], dst_ref=recv_left.at[l],
                send_sem=s_own_send.at[l, 1], recv_sem=s_own_recv.at[l, 0],
                device_id=(right,), device_id_type=pl.DeviceIdType.MESH)
            dL.start()
            dR.start()
            d_toL.append(dL)
            d_toR.append(dR)

        def packed_term(xb, p):
            h = jnp.maximum(
                jnp.dot(xb, p[0:D, :], preferred_element_type=jnp.float32),
                0.0)
            hb = h.astype(jnp.bfloat16)
            return (jnp.dot(hb[:, 0:D], p[D:2 * D, 0:D],
                            preferred_element_type=jnp.float32) +
                    jnp.dot(hb[:, D:2 * D], p[D:2 * D, D:2 * D],
                            preferred_element_type=jnp.float32))

        def own_term(xv, l):
            h = jnp.maximum(
                jnp.dot(xv, win_refs[l][...],
                        preferred_element_type=jnp.float32), 0.0)
            return jnp.dot(h, wout_refs[l][...],
                           preferred_element_type=jnp.float32)

        x_val = x_ref[...]
        acc = own_term(x_val, 0)
        d_relL, d_relR = [], []
        for l in range(NL):
            xb = x_val.astype(jnp.bfloat16)
            d_toR[l].wait_recv()
            rR = pltpu.make_async_remote_copy(
                src_ref=recv_left.at[l, pl.ds(0, D), :],
                dst_ref=recv_diag.at[l, pl.ds(0, D), :],
                send_sem=s_rel_send.at[l, 1], recv_sem=s_rel_recv.at[l, 0],
                device_id=(right,), device_id_type=pl.DeviceIdType.MESH)
            rR.start()
            d_relR.append(rR)
            acc = acc + packed_term(xb, recv_left[l])

            d_toL[l].wait_recv()
            rL = pltpu.make_async_remote_copy(
                src_ref=recv_right.at[l, pl.ds(D, D), :],
                dst_ref=recv_diag.at[l, pl.ds(D, D), :],
                send_sem=s_rel_send.at[l, 0], recv_sem=s_rel_recv.at[l, 1],
                device_id=(left,), device_id_type=pl.DeviceIdType.MESH)
            rL.start()
            d_relL.append(rL)
            acc = acc + packed_term(xb, recv_right[l])

            d_relR[l].wait_recv()
            d_relL[l].wait_recv()
            acc = acc + packed_term(xb, recv_diag[l])

            x_val = acc
            if l + 1 < NL:
                acc = own_term(x_val, l + 1)

        out_ref[pl.ds(me * B, B), :] = x_val.astype(jnp.bfloat16)
        d_agL = pltpu.make_async_remote_copy(
            src_ref=out_ref.at[pl.ds(me * B, B), :],
            dst_ref=out_ref.at[pl.ds(me * B, B), :],
            send_sem=s_ag_own_send.at[0], recv_sem=s_ag_own_recv.at[1],
            device_id=(left,), device_id_type=pl.DeviceIdType.MESH)
        d_agR = pltpu.make_async_remote_copy(
            src_ref=out_ref.at[pl.ds(me * B, B), :],
            dst_ref=out_ref.at[pl.ds(me * B, B), :],
            send_sem=s_ag_own_send.at[1], recv_sem=s_ag_own_recv.at[0],
            device_id=(right,), device_id_type=pl.DeviceIdType.MESH)
        d_agL.start()
        d_agR.start()

        d_agR.wait_recv()
        d_agrelR = pltpu.make_async_remote_copy(
            src_ref=out_ref.at[pl.ds(left * B, HB), :],
            dst_ref=out_ref.at[pl.ds(left * B, HB), :],
            send_sem=s_ag_rel_send.at[1], recv_sem=s_ag_rel_recv.at[0],
            device_id=(right,), device_id_type=pl.DeviceIdType.MESH)
        d_agrelR.start()

        d_agL.wait_recv()
        d_agrelL = pltpu.make_async_remote_copy(
            src_ref=out_ref.at[pl.ds(right * B + HB, HB), :],
            dst_ref=out_ref.at[pl.ds(right * B + HB, HB), :],
            send_sem=s_ag_rel_send.at[0], recv_sem=s_ag_rel_recv.at[1],
            device_id=(left,), device_id_type=pl.DeviceIdType.MESH)
        d_agrelL.start()

        d_agrelR.wait_recv()
        d_agrelL.wait_recv()

        for d in d_toL + d_toR + d_relL + d_relR:
            d.wait_send()
        for d in (d_agL, d_agR, d_agrelR, d_agrelL):
            d.wait_send()

    return pl.pallas_call(
        body,
        out_shape=jax.ShapeDtypeStruct((N_DEV * B, D), jnp.bfloat16),
        in_specs=[pl.BlockSpec(memory_space=pltpu.VMEM)] * 7,
        out_specs=pl.BlockSpec(memory_space=pltpu.VMEM),
        scratch_shapes=[
            pltpu.VMEM((NL, HS, HS), jnp.bfloat16),
            pltpu.VMEM((NL, HS, HS), jnp.bfloat16),
            pltpu.VMEM((NL, HS, HS), jnp.bfloat16),
            pltpu.VMEM((NL, HS, HS), jnp.bfloat16),
            pltpu.SemaphoreType.DMA((NL, 2)),
            pltpu.SemaphoreType.DMA((NL, 2)),
            pltpu.SemaphoreType.DMA((NL, 2)),
            pltpu.SemaphoreType.DMA((NL, 2)),
            pltpu.SemaphoreType.DMA((2,)),
            pltpu.SemaphoreType.DMA((2,)),
            pltpu.SemaphoreType.DMA((2,)),
            pltpu.SemaphoreType.DMA((2,)),
        ],
        compiler_params=pltpu.CompilerParams(collective_id=0),
    )(x, Win0, Wout0, Win1, Wout1, Win2, Wout2)
